# baseline (speedup 1.0000x reference)
import functools

import jax
import jax.numpy as jnp
from jax import lax
from jax.experimental import pallas as pl
from jax.experimental.pallas import tpu as pltpu
from jax.experimental.pallas import tpu_sc as plsc

SEQ = 32768
VOCAB = 5
D = 384
NC = 2
NS = 16
NW = NC * NS
BPW = SEQ // NW
C = 128
NCHUNK = BPW // C

_mesh = plsc.VectorSubcoreMesh(core_axis_name="c", subcore_axis_name="s")


@functools.partial(
    pl.kernel,
    mesh=_mesh,
    out_type=jax.ShapeDtypeStruct((SEQ, D), jnp.float32),
    scratch_types=[
        pltpu.VMEM((BPW,), jnp.int32),
        pltpu.VMEM((2, C, D), jnp.float32),
        pltpu.SemaphoreType.DMA,
        pltpu.SemaphoreType.DMA,
    ],
)
def _embed(ids_hbm, table_hbm, out_hbm, idx_v, rows_v, gsem, wsem):
    wid = lax.axis_index("s") * NC + lax.axis_index("c")
    base = wid * BPW
    pltpu.sync_copy(ids_hbm.at[pl.ds(base, BPW)], idx_v)

    writes = []
    for k in range(NCHUNK):
        if k >= 2:
            writes[k - 2].wait()
        buf = rows_v.at[k % 2]
        gather = pltpu.async_copy(
            table_hbm.at[idx_v.at[pl.ds(k * C, C)]], buf, gsem)
        gather.wait()
        writes.append(
            pltpu.async_copy(buf, out_hbm.at[pl.ds(base + k * C, C)], wsem))
    writes[-2].wait()
    writes[-1].wait()


def kernel(ids, table):
    return _embed(ids.astype(jnp.int32), table)


# speedup vs baseline: 5.3883x; 5.3883x over previous
import functools

import jax
import jax.numpy as jnp
from jax import lax
from jax.experimental import pallas as pl
from jax.experimental.pallas import tpu as pltpu
from jax.experimental.pallas import tpu_sc as plsc

SEQ = 32768
VOCAB = 5
D = 384
NC = 2
NS = 16
NW = NC * NS
BPW = SEQ // NW
C = 128
NCHUNK = BPW // C

_mesh = plsc.VectorSubcoreMesh(core_axis_name="c", subcore_axis_name="s")


@functools.partial(
    pl.kernel,
    mesh=_mesh,
    out_type=(
        jax.ShapeDtypeStruct((SEQ, D), jnp.float32),
        jax.ShapeDtypeStruct((NW, VOCAB, D), jnp.float32),
    ),
    scratch_types=[
        pltpu.VMEM((BPW,), jnp.int32),
        pltpu.VMEM((2, C, D), jnp.float32),
        pltpu.VMEM((VOCAB, D), jnp.float32),
        pltpu.SemaphoreType.DMA,
        pltpu.SemaphoreType.DMA,
    ],
)
def _embed(ids_hbm, table_hbm, out_hbm, rep_hbm, idx_v, rows_v, table_v,
           gsem, wsem):
    wid = lax.axis_index("s") * NC + lax.axis_index("c")
    base = wid * BPW

    pltpu.sync_copy(table_hbm, table_v)
    pltpu.sync_copy(table_v, rep_hbm.at[wid])
    pltpu.sync_copy(ids_hbm.at[pl.ds(base, BPW)], idx_v)

    writes = []
    for k in range(NCHUNK):
        if k >= 2:
            writes[k - 2].wait()
        buf = rows_v.at[k % 2]
        gather = pltpu.async_copy(
            rep_hbm.at[wid].at[idx_v.at[pl.ds(k * C, C)]], buf, gsem)
        gather.wait()
        writes.append(
            pltpu.async_copy(buf, out_hbm.at[pl.ds(base + k * C, C)], wsem))
    writes[-2].wait()
    writes[-1].wait()


def kernel(ids, table):
    out, _ = _embed(ids.astype(jnp.int32), table)
    return out


# speedup vs baseline: 5.3901x; 1.0003x over previous
import functools

import jax
import jax.numpy as jnp
from jax import lax
from jax.experimental import pallas as pl
from jax.experimental.pallas import tpu as pltpu
from jax.experimental.pallas import tpu_sc as plsc

SEQ = 32768
VOCAB = 5
D = 384
NC = 2
NS = 16
NW = NC * NS
BPW = SEQ // NW
C = 64
NCHUNK = BPW // C
NBUF = 4

_mesh = plsc.VectorSubcoreMesh(core_axis_name="c", subcore_axis_name="s")


@functools.partial(
    pl.kernel,
    mesh=_mesh,
    out_type=(
        jax.ShapeDtypeStruct((SEQ, D), jnp.float32),
        jax.ShapeDtypeStruct((NW, VOCAB, D), jnp.float32),
    ),
    scratch_types=[
        pltpu.VMEM((BPW,), jnp.int32),
        pltpu.VMEM((NBUF, C, D), jnp.float32),
        pltpu.VMEM((VOCAB, D), jnp.float32),
        pltpu.SemaphoreType.DMA,
        pltpu.SemaphoreType.DMA,
    ],
)
def _embed(ids_hbm, table_hbm, out_hbm, rep_hbm, idx_v, rows_v, table_v,
           gsem, wsem):
    wid = lax.axis_index("s") * NC + lax.axis_index("c")
    base = wid * BPW

    pltpu.sync_copy(table_hbm, table_v)
    pltpu.sync_copy(table_v, rep_hbm.at[wid])
    pltpu.sync_copy(ids_hbm.at[pl.ds(base, BPW)], idx_v)

    def fire_gather(j):
        return pltpu.async_copy(
            rep_hbm.at[wid].at[idx_v.at[pl.ds(j * C, C)]],
            rows_v.at[j % NBUF], gsem)

    gathers = [fire_gather(0), fire_gather(1)]
    writes = []
    for k in range(NCHUNK):
        gathers[k].wait()
        writes.append(
            pltpu.async_copy(rows_v.at[k % NBUF],
                             out_hbm.at[pl.ds(base + k * C, C)], wsem))
        j = k + 2
        if j < NCHUNK:
            if j >= NBUF:
                writes[j - NBUF].wait()
            gathers.append(fire_gather(j))
    for w in writes[-NBUF:]:
        w.wait()


def kernel(ids, table):
    out, _ = _embed(ids.astype(jnp.int32), table)
    return out


# speedup vs baseline: 14.4008x; 2.6717x over previous
import functools

import jax
import jax.numpy as jnp
from jax import lax
from jax.experimental import pallas as pl
from jax.experimental.pallas import tpu as pltpu
from jax.experimental.pallas import tpu_sc as plsc

SEQ = 32768
VOCAB = 5
D = 384
LANES = 16
NC = 2
NS = 16
NW = NC * NS
BPW = SEQ // NW
C = 128
NCHUNK = BPW // C
NBUF = 2

_mesh = plsc.VectorSubcoreMesh(core_axis_name="c", subcore_axis_name="s")


@functools.partial(
    pl.kernel,
    mesh=_mesh,
    compiler_params=pltpu.CompilerParams(needs_layout_passes=False),
    out_type=jax.ShapeDtypeStruct((SEQ, D), jnp.float32),
    scratch_types=[
        pltpu.VMEM((BPW,), jnp.int32),
        pltpu.VMEM((NBUF, C, D), jnp.float32),
        pltpu.VMEM((VOCAB * D,), jnp.float32),
        pltpu.SemaphoreType.DMA,
    ],
)
def _embed(ids_hbm, table_hbm, out_hbm, idx_v, rows_v, table_v, wsem):
    wid = lax.axis_index("s") * NC + lax.axis_index("c")
    base = wid * BPW

    pltpu.sync_copy(table_hbm, table_v)
    pltpu.sync_copy(ids_hbm.at[pl.ds(base, BPW)], idx_v)

    coliota = lax.iota(jnp.int32, LANES)

    def drain(b):
        pltpu.make_async_copy(
            rows_v.at[b], out_hbm.at[pl.ds(base, C)], wsem).wait()

    @pl.loop(0, NCHUNK, step=NBUF)
    def _round(k0):
        for b in range(NBUF):
            k = k0 + b

            @pl.when(k0 > 0)
            def _():
                drain(b)

            buf = rows_v.at[b]

            @plsc.parallel_loop(0, C, unroll=2)
            def _row(i, buf=buf, k=k):
                r = lax.rem(i, LANES)
                g16 = i - r
                ids16 = idx_v[pl.ds(k * C + g16, LANES)]
                rid16 = jnp.take(ids16, jnp.full((LANES,), 0, jnp.int32) + r)
                off = rid16 * D + coliota
                vals = [
                    plsc.load_gather(
                        table_v.at[pl.ds(j * LANES, VOCAB * D - j * LANES)],
                        [off])
                    for j in range(D // LANES)
                ]
                for j in range(D // LANES):
                    buf[i, pl.ds(j * LANES, LANES)] = vals[j]

            pltpu.async_copy(buf, out_hbm.at[pl.ds(base + k * C, C)], wsem)

    for b in range(NBUF):
        drain(b)


def kernel(ids, table):
    return _embed(ids.astype(jnp.int32), table.reshape(VOCAB * D))
